# baseline (speedup 1.0000x reference)
import functools

import jax
import jax.numpy as jnp
from jax import lax
from jax.experimental import pallas as pl
from jax.experimental.pallas import tpu as pltpu
from jax.experimental.pallas import tpu_sc as plsc

NC = 2
NS = 16
NW = NC * NS

V = 1000000
D = 64
N_ROWS = 4096
ROW_LEN = 200

FULL_UNITS = V // 128
TAIL = V - FULL_UNITS * 128
K_PER_W = FULL_UNITS // NW
K_REM = FULL_UNITS % NW
I_BLK = N_ROWS // NW


def _iota16():
    return lax.iota(jnp.int32, 16)


def kernel(token_ids, embedding_table):
    assert token_ids.shape == (N_ROWS, ROW_LEN)
    assert embedding_table.shape == (V, D)

    mesh = plsc.VectorSubcoreMesh(core_axis_name="c", subcore_axis_name="s")

    @functools.partial(
        pl.kernel,
        mesh=mesh,
        out_type=jax.ShapeDtypeStruct((V * D // 128, 128), jnp.float32),
        scratch_types=[
            pltpu.VMEM((2, 8, 8, 128), jnp.float32),
            pltpu.VMEM((2, 64, 128), jnp.float32),
            pltpu.VMEM((8, 8, TAIL), jnp.float32),
            pltpu.VMEM((TAIL // 2, 128), jnp.float32),
            pltpu.SemaphoreType.DMA((2,)),
            pltpu.SemaphoreType.DMA((2,)),
        ],
        compiler_params=pltpu.CompilerParams(
            use_tc_tiling_on_sc=True, needs_layout_passes=False
        ),
    )
    def k1(tt_hbm, x_hbm, vin, vout, tin, tout, isem, osem):
        wid = lax.axis_index("s") * NC + lax.axis_index("c")
        ii = _iota16()
        avecs = [(ii + 16 * m) // 8 for m in range(4)]
        cvecs = [(ii + 16 * m) % 8 for m in range(4)]

        def t0_of(k):
            return pl.multiple_of((wid + NW * k) * 128, 128)

        def in_start(k, s):
            t0 = t0_of(k)
            for a in range(8):
                pltpu.async_copy(
                    tt_hbm.at[pl.ds(8 * a, 8), pl.ds(t0, 128)],
                    vin.at[s, a], isem.at[s],
                )

        def in_wait(k, s):
            t0 = t0_of(k)
            for a in range(8):
                pltpu.make_async_copy(
                    tt_hbm.at[pl.ds(8 * a, 8), pl.ds(t0, 128)],
                    vin.at[s, a], isem.at[s],
                ).wait()

        def out_start(k, s):
            base = pl.multiple_of(t0_of(k) // 2, 64)
            pltpu.async_copy(vout.at[s], x_hbm.at[pl.ds(base, 64)], osem.at[s])

        def out_wait(k, s):
            base = pl.multiple_of(t0_of(k) // 2, 64)
            pltpu.make_async_copy(
                vout.at[s], x_hbm.at[pl.ds(base, 64)], osem.at[s]
            ).wait()

        def transpose(s):
            @plsc.parallel_loop(0, 64, unroll=8)
            def _rows(p):
                for h in range(2):
                    tv = jnp.full((16,), 2 * p + h, jnp.int32)
                    for m in range(4):
                        g = plsc.load_gather(
                            vin.at[s], [avecs[m], cvecs[m], tv]
                        )
                        vout[s, p, pl.ds(64 * h + 16 * m, 16)] = g

        in_start(0, 0)

        @pl.loop(0, K_PER_W // 2)
        def _units(k2):
            for par in range(2):
                k = 2 * k2 + par
                s = par

                @pl.when(jnp.logical_or(k + 1 < K_PER_W, wid < K_REM))
                def _prefetch():
                    in_start(k + 1, 1 - s)

                in_wait(k, s)

                @pl.when(k >= 2)
                def _recycle():
                    out_wait(k - 2, s)

                transpose(s)
                out_start(k, s)

        @pl.when(wid < K_REM)
        def _extra():
            k = K_PER_W
            in_wait(k, 0)
            out_wait(k - 2, 0)
            transpose(0)
            out_start(k, 0)

        @pl.when(wid < K_REM)
        def _drain0a():
            out_wait(K_PER_W, 0)

        @pl.when(wid >= K_REM)
        def _drain0b():
            out_wait(K_PER_W - 2, 0)

        out_wait(K_PER_W - 1, 1)

        @pl.when(wid == K_REM)
        def _tail():
            t0 = FULL_UNITS * 128
            for a in range(8):
                pltpu.async_copy(
                    tt_hbm.at[pl.ds(8 * a, 8), pl.ds(t0, TAIL)],
                    tin.at[a], isem.at[0],
                )
            for a in range(8):
                pltpu.make_async_copy(
                    tt_hbm.at[pl.ds(8 * a, 8), pl.ds(t0, TAIL)],
                    tin.at[a], isem.at[0],
                ).wait()

            @plsc.parallel_loop(0, TAIL // 2, unroll=4)
            def _trows(p):
                for h in range(2):
                    tv = jnp.full((16,), 2 * p + h, jnp.int32)
                    for m in range(4):
                        g = plsc.load_gather(tin, [avecs[m], cvecs[m], tv])
                        tout[p, pl.ds(64 * h + 16 * m, 16)] = g

            pltpu.async_copy(
                tout, x_hbm.at[pl.ds(t0 // 2, TAIL // 2)], osem.at[0]
            )
            pltpu.make_async_copy(
                tout, x_hbm.at[pl.ds(t0 // 2, TAIL // 2)], osem.at[0]
            ).wait()

    @functools.partial(
        pl.kernel,
        mesh=mesh,
        out_type=jax.ShapeDtypeStruct(
            (ROW_LEN, 8, N_ROWS // 128, 8, 128), jnp.float32
        ),
        scratch_types=[
            pltpu.VMEM((I_BLK, ROW_LEN), jnp.int32),
            pltpu.VMEM((ROW_LEN, I_BLK), jnp.int32),
            pltpu.VMEM((2, 128, D), jnp.float32),
            pltpu.VMEM((2, D, 128), jnp.float32),
            pltpu.SemaphoreType.DMA((2,)),
            pltpu.SemaphoreType.DMA((2,)),
        ],
        compiler_params=pltpu.CompilerParams(
            use_tc_tiling_on_sc=False, needs_layout_passes=False
        ),
    )
    def k2(x_hbm, ids_hbm, o_hbm, idv, idt, rows, tbuf, gsem, osem):
        wid = lax.axis_index("s") * NC + lax.axis_index("c")
        ii = _iota16()

        pltpu.sync_copy(ids_hbm.at[pl.ds(I_BLK * wid, I_BLK), :], idv)

        @plsc.parallel_loop(0, ROW_LEN, unroll=4)
        def _idrows(j):
            jv = jnp.full((16,), j, jnp.int32)
            for m in range(8):
                g = plsc.load_gather(idv, [ii + 16 * m, jv])
                idt[j, pl.ds(16 * m, 16)] = g

        def g_start(j, s):
            pltpu.async_copy(x_hbm.at[idt.at[j]], rows.at[s], gsem.at[s])

        def g_wait(j, s):
            pltpu.make_async_copy(
                x_hbm.at[idt.at[j]], rows.at[s], gsem.at[s]
            ).wait()

        def o_start(j, s):
            for ct in range(8):
                pltpu.async_copy(
                    tbuf.at[s, pl.ds(8 * ct, 8)], o_hbm.at[j, ct, wid],
                    osem.at[s],
                )

        def o_wait(j, s):
            for ct in range(8):
                pltpu.make_async_copy(
                    tbuf.at[s, pl.ds(8 * ct, 8)], o_hbm.at[j, ct, wid],
                    osem.at[s],
                ).wait()

        def transpose(s):
            @plsc.parallel_loop(0, D, unroll=8)
            def _cols(c):
                cv = jnp.full((16,), c, jnp.int32)
                for m in range(8):
                    g = plsc.load_gather(rows.at[s], [ii + 16 * m, cv])
                    tbuf[s, c, pl.ds(16 * m, 16)] = g

        g_start(0, 0)

        @pl.loop(0, ROW_LEN // 2)
        def _panels(j2):
            for par in range(2):
                j = 2 * j2 + par
                s = par

                @pl.when(j + 1 < ROW_LEN)
                def _prefetch():
                    g_start(j + 1, 1 - s)

                g_wait(j, s)

                @pl.when(j >= 2)
                def _recycle():
                    o_wait(j - 2, s)

                transpose(s)
                o_start(j, s)

        o_wait(ROW_LEN - 2, 0)
        o_wait(ROW_LEN - 1, 1)

    ids = token_ids.astype(jnp.int32)
    tt = embedding_table.T
    x = k1(tt)
    xv = x.reshape(V, D)
    o2 = k2(xv, ids)
    return o2.transpose(2, 4, 0, 1, 3).reshape(N_ROWS, ROW_LEN, D)


# speedup vs baseline: 1.2133x; 1.2133x over previous
import functools

import jax
import jax.numpy as jnp
from jax import lax
from jax.experimental import pallas as pl
from jax.experimental.pallas import tpu as pltpu
from jax.experimental.pallas import tpu_sc as plsc

NC = 2
NS = 16
NW = NC * NS

NBUF = 4
LOOKAHEAD = 3


@jax.jit
def kernel(token_ids, embedding_table):
    n_rows, row_len = token_ids.shape
    d = embedding_table.shape[1]
    ids = token_ids.astype(jnp.int32)
    assert n_rows % (NW * NBUF) == 0 and row_len % 8 == 0
    rows_per_w = n_rows // NW
    n_chunks = rows_per_w

    mesh = plsc.VectorSubcoreMesh(core_axis_name="c", subcore_axis_name="s")

    @functools.partial(
        pl.kernel,
        mesh=mesh,
        out_type=jax.ShapeDtypeStruct((n_rows, row_len, d), jnp.float32),
        scratch_types=[
            pltpu.VMEM((rows_per_w, row_len), jnp.int32),
            pltpu.VMEM((NBUF, row_len, d), jnp.float32),
            pltpu.SemaphoreType.DMA((NBUF,)),
            pltpu.SemaphoreType.DMA((NBUF,)),
        ],
        compiler_params=pltpu.CompilerParams(use_tc_tiling_on_sc=False),
    )
    def emb(table_hbm, idx_hbm, out_hbm, idx_v, rows_v, gsem, osem):
        wid = lax.axis_index("s") * NC + lax.axis_index("c")
        base = wid * rows_per_w
        pltpu.sync_copy(idx_hbm.at[pl.ds(base, rows_per_w), :], idx_v)

        def g_ref(c):
            return table_hbm.at[idx_v.at[c]]

        def o_ref(c):
            return out_hbm.at[base + c]

        def gather_start(c, bf):
            pltpu.async_copy(g_ref(c), rows_v.at[bf], gsem.at[bf])

        def gather_wait(c, bf):
            pltpu.make_async_copy(g_ref(c), rows_v.at[bf], gsem.at[bf]).wait()

        def out_start(c, bf):
            pltpu.async_copy(rows_v.at[bf], o_ref(c), osem.at[bf])

        def out_wait(c, bf):
            pltpu.make_async_copy(rows_v.at[bf], o_ref(c), osem.at[bf]).wait()

        for j in range(LOOKAHEAD):
            gather_start(j, j)

        gather_wait(0, 0)
        out_start(0, 0)
        gather_start(LOOKAHEAD, LOOKAHEAD % NBUF)
        for bf in range(1, NBUF):
            c = bf
            gather_wait(c, bf)
            out_start(c, bf)
            nf = (bf + LOOKAHEAD) % NBUF
            out_wait(c - 1, nf)
            gather_start(c + LOOKAHEAD, nf)

        @pl.loop(1, n_chunks // NBUF - 1)
        def _steady(i):
            c0 = i * NBUF
            for bf in range(NBUF):
                c = c0 + bf
                gather_wait(c, bf)
                out_start(c, bf)
                nf = (bf + LOOKAHEAD) % NBUF
                out_wait(c - 1, nf)
                gather_start(c + LOOKAHEAD, nf)

        c0 = n_chunks - NBUF
        gather_wait(c0, 0)
        out_start(c0, 0)
        out_wait(c0 - 1, LOOKAHEAD % NBUF)
        gather_start(c0 + LOOKAHEAD, LOOKAHEAD % NBUF)
        for bf in range(1, NBUF):
            c = c0 + bf
            gather_wait(c, bf)
            out_start(c, bf)
        for bf in range(NBUF):
            out_wait(c0 + bf, bf)

    return emb(embedding_table, ids)


# speedup vs baseline: 1.6178x; 1.3333x over previous
import functools

import jax
import jax.numpy as jnp
from jax import lax
from jax.experimental import pallas as pl
from jax.experimental.pallas import tpu as pltpu
from jax.experimental.pallas import tpu_sc as plsc

NC = 2
NS = 16
NW = NC * NS

NBUF = 4
LOOKAHEAD = 3


@jax.jit
def kernel(token_ids, embedding_table):
    n_rows, row_len = token_ids.shape
    d = embedding_table.shape[1]
    ids = token_ids.astype(jnp.int32)
    assert n_rows % (NW * NBUF) == 0 and row_len % 8 == 0
    rows_per_w = n_rows // NW
    n_chunks = rows_per_w

    mesh = plsc.VectorSubcoreMesh(core_axis_name="c", subcore_axis_name="s")

    @functools.partial(
        pl.kernel,
        mesh=mesh,
        out_type=jax.ShapeDtypeStruct((n_rows, row_len, 128), jnp.float32),
        scratch_types=[
            pltpu.VMEM((rows_per_w, row_len), jnp.int32),
            pltpu.VMEM((NBUF, row_len, d), jnp.float32),
            pltpu.SemaphoreType.DMA((NBUF,)),
            pltpu.SemaphoreType.DMA((NBUF,)),
        ],
        compiler_params=pltpu.CompilerParams(use_tc_tiling_on_sc=False),
    )
    def emb(table_hbm, idx_hbm, out_hbm, idx_v, rows_v, gsem, osem):
        wid = lax.axis_index("s") * NC + lax.axis_index("c")
        base = wid * rows_per_w
        pltpu.sync_copy(idx_hbm.at[pl.ds(base, rows_per_w), :], idx_v)

        def g_ref(c):
            return table_hbm.at[idx_v.at[c]]

        def o_ref(c):
            return out_hbm.at[base + c, :, pl.ds(0, d)]

        def gather_start(c, bf):
            pltpu.async_copy(g_ref(c), rows_v.at[bf], gsem.at[bf])

        def gather_wait(c, bf):
            pltpu.make_async_copy(g_ref(c), rows_v.at[bf], gsem.at[bf]).wait()

        def out_start(c, bf):
            pltpu.async_copy(rows_v.at[bf], o_ref(c), osem.at[bf])

        def out_wait(c, bf):
            pltpu.make_async_copy(rows_v.at[bf], o_ref(c), osem.at[bf]).wait()

        for j in range(LOOKAHEAD):
            gather_start(j, j)

        gather_wait(0, 0)
        out_start(0, 0)
        gather_start(LOOKAHEAD, LOOKAHEAD % NBUF)
        for bf in range(1, NBUF):
            c = bf
            gather_wait(c, bf)
            out_start(c, bf)
            nf = (bf + LOOKAHEAD) % NBUF
            out_wait(c - 1, nf)
            gather_start(c + LOOKAHEAD, nf)

        @pl.loop(1, n_chunks // NBUF - 1)
        def _steady(i):
            c0 = i * NBUF
            for bf in range(NBUF):
                c = c0 + bf
                gather_wait(c, bf)
                out_start(c, bf)
                nf = (bf + LOOKAHEAD) % NBUF
                out_wait(c - 1, nf)
                gather_start(c + LOOKAHEAD, nf)

        c0 = n_chunks - NBUF
        gather_wait(c0, 0)
        out_start(c0, 0)
        out_wait(c0 - 1, LOOKAHEAD % NBUF)
        gather_start(c0 + LOOKAHEAD, LOOKAHEAD % NBUF)
        for bf in range(1, NBUF):
            c = c0 + bf
            gather_wait(c, bf)
            out_start(c, bf)
        for bf in range(NBUF):
            out_wait(c0 + bf, bf)

    out = emb(embedding_table, ids)
    return out[:, :, :d]
